# Initial kernel scaffold; baseline (speedup 1.0000x reference)
import jax
import jax.numpy as jnp
from jax.experimental import pallas as pl


def kernel(x, prev_h, edge_index, lambda_Q, lambda_B):
    raise NotImplementedError("write your pallas kernel here")


# speedup vs baseline: 35.6867x; 35.6867x over previous
import functools

import jax
import jax.numpy as jnp
from jax import lax
from jax.experimental import pallas as pl
from jax.experimental.pallas import tpu as pltpu
from jax.experimental.pallas import tpu_sc as plsc

N = 10000
E = 160000
C = 16
M = 32
G = 16
CG = C * G

NC = 2
NS = 16
HALF = N // NC
TRASH = HALF
ACC_ROWS = 5120
KC = 128
CHUNKS_PER_SUB = 79
E_PAD = CHUNKS_PER_SUB * NS * KC
EDGES_PER_SUB = CHUNKS_PER_SUB * KC


def _sc_segment_sum(ph2, dst_pad, src_pad):
  mesh = plsc.VectorSubcoreMesh(core_axis_name="c", subcore_axis_name="s")

  @functools.partial(
      pl.kernel,
      out_type=(
          jax.ShapeDtypeStruct((N, CG), jnp.float32),
          jax.ShapeDtypeStruct((N, 16), jnp.float32),
      ),
      mesh=mesh,
      compiler_params=pltpu.CompilerParams(use_tc_tiling_on_sc=False),
      scratch_types=[
          pltpu.VMEM_SHARED((ACC_ROWS, CG), jnp.float32),
          pltpu.VMEM_SHARED((ACC_ROWS, 16), jnp.float32),
          pltpu.VMEM((1, KC), jnp.int32),
          pltpu.VMEM((1, KC), jnp.int32),
          pltpu.VMEM((1, KC), jnp.int32),
          pltpu.VMEM((KC, CG), jnp.float32),
          pltpu.VMEM((KC, 16), jnp.float32),
          pltpu.VMEM((KC, 16), jnp.float32),
          pltpu.SemaphoreType.DMA,
      ],
  )
  def body(ph_hbm, dst_hbm, src_hbm, sums_hbm, cnts_hbm,
           acc, cacc, dstv, srcv, ldv, rows, ones16, z16, sem):
    cid = lax.axis_index("c")
    sid = lax.axis_index("s")
    base = cid * HALF

    def fill_row(i, _):
      ones16[i, :] = jnp.full((16,), 1.0, jnp.float32)
      z16[i, :] = jnp.zeros((16,), jnp.float32)
      return 0
    lax.fori_loop(0, KC, fill_row, 0)

    def zero_rows(i, _):
      def zero_seg(j, _):
        rows[i, pl.ds(j * 16, 16)] = jnp.zeros((16,), jnp.float32)
        return 0
      lax.fori_loop(0, CG // 16, zero_seg, 0)
      return 0
    lax.fori_loop(0, KC, zero_rows, 0)

    nz = ACC_ROWS // KC
    for k in range((nz + NS - 1) // NS):
      q = sid + NS * k
      @pl.when(q < nz)
      def _():
        pltpu.sync_copy(rows, acc.at[pl.ds(q * KC, KC)])
        pltpu.sync_copy(z16, cacc.at[pl.ds(q * KC, KC)])
    plsc.subcore_barrier()

    ebase = sid * EDGES_PER_SUB

    def chunk(t, _):
      eoff = ebase + t * KC
      pltpu.sync_copy(dst_hbm.at[pl.ds(eoff, KC)], dstv.at[0])
      pltpu.sync_copy(src_hbm.at[pl.ds(eoff, KC)], srcv.at[0])
      for j in range(KC // 16):
        d = dstv[0, pl.ds(j * 16, 16)]
        l = d - base
        ok = (l >= 0) & (l < HALF)
        ldv[0, pl.ds(j * 16, 16)] = jnp.where(ok, l, TRASH)
      pltpu.async_copy(ph_hbm.at[srcv.at[0]], rows, sem).wait()
      pltpu.sync_copy(rows, acc.at[ldv.at[0]], add=True)
      pltpu.sync_copy(ones16, cacc.at[ldv.at[0]], add=True)
      return 0

    lax.fori_loop(0, CHUNKS_PER_SUB, chunk, 0)
    plsc.subcore_barrier()

    nq = HALF // 200
    for k in range((nq + NS - 1) // NS):
      q = sid + NS * k
      @pl.when(q < nq)
      def _():
        r0 = q * 200
        pltpu.sync_copy(acc.at[pl.ds(r0, 200)],
                        sums_hbm.at[pl.ds(base + r0, 200)])
        pltpu.sync_copy(cacc.at[pl.ds(r0, 200)],
                        cnts_hbm.at[pl.ds(base + r0, 200)])

  return body(ph2, dst_pad, src_pad)


def _tc_body(sums_ref, cnts_ref, x_ref, lamqt_ref, lamb2_ref,
             logtot_ref, post_ref):
  f32 = jnp.float32
  lamqt = lamqt_ref[...]
  qm = jnp.max(lamqt, axis=1, keepdims=True)
  qe = jnp.exp(lamqt - qm)
  qs = qe / jnp.sum(qe, axis=1, keepdims=True)
  r16 = lax.broadcasted_iota(jnp.int32, (C, CG), 0)
  c256 = lax.broadcasted_iota(jnp.int32, (C, CG), 1)
  e16 = (lax.div(c256, G) == r16).astype(f32)
  qsel = jnp.dot(qs, e16, preferred_element_type=f32)
  rr = lax.broadcasted_iota(jnp.int32, (CG, CG), 0)
  cc = lax.broadcasted_iota(jnp.int32, (CG, CG), 1)
  w = qsel * (lax.rem(rr, G) == lax.rem(cc, G)).astype(f32)

  lamb2 = lamb2_ref[...]
  bm = jnp.max(lamb2, axis=0, keepdims=True)
  be = jnp.exp(lamb2 - bm)
  bs = be / jnp.sum(be, axis=0, keepdims=True)

  nb = sums_ref.shape[0]
  cm = jnp.maximum(cnts_ref[...][:, 0:1], 1.0)
  aggr = sums_ref[...] / cm

  qa = jnp.dot(aggr, w, preferred_element_type=f32)

  xb = x_ref[...]
  mio = lax.broadcasted_iota(jnp.int32, (nb, M), 1)
  oh = (xb == mio).astype(f32)
  bn = jnp.dot(oh, bs, preferred_element_type=f32)

  tmp = bn * qa
  sr = lax.broadcasted_iota(jnp.int32, (CG, G), 0)
  sc = lax.broadcasted_iota(jnp.int32, (CG, G), 1)
  s_mat = (lax.rem(sr, G) == sc).astype(f32)
  total = jnp.dot(tmp, s_mat, preferred_element_type=f32)
  tr = lax.broadcasted_iota(jnp.int32, (G, CG), 0)
  tc = lax.broadcasted_iota(jnp.int32, (G, CG), 1)
  st_mat = (tr == lax.rem(tc, G)).astype(f32)
  totb = jnp.dot(total, st_mat, preferred_element_type=f32)

  logtot_ref[...] = jnp.log(total)
  post_ref[...] = tmp / totb


def _tc_dense(sums, cnts, x2, lamqt, lamb2):
  nb = 1000
  grid = N // nb
  return pl.pallas_call(
      _tc_body,
      grid=(grid,),
      in_specs=[
          pl.BlockSpec((nb, CG), lambda i: (i, 0)),
          pl.BlockSpec((nb, 16), lambda i: (i, 0)),
          pl.BlockSpec((nb, 1), lambda i: (i, 0)),
          pl.BlockSpec((CG, C), lambda i: (0, 0)),
          pl.BlockSpec((M, CG), lambda i: (0, 0)),
      ],
      out_specs=[
          pl.BlockSpec((nb, G), lambda i: (i, 0)),
          pl.BlockSpec((nb, CG), lambda i: (i, 0)),
      ],
      out_shape=[
          jax.ShapeDtypeStruct((N, G), jnp.float32),
          jax.ShapeDtypeStruct((N, CG), jnp.float32),
      ],
  )(sums, cnts, x2, lamqt, lamb2)


def kernel(x, prev_h, edge_index, lambda_Q, lambda_B):
  ph2 = prev_h.reshape(N, CG)
  dst = edge_index[0]
  src = edge_index[1]
  pad = E_PAD - E
  dst_pad = jnp.concatenate([dst, jnp.full((pad,), -1, dst.dtype)])
  src_pad = jnp.concatenate([src, jnp.zeros((pad,), src.dtype)])

  sums, cnts = _sc_segment_sum(ph2, dst_pad.astype(jnp.int32),
                               src_pad.astype(jnp.int32))

  lamqt = jnp.transpose(lambda_Q, (1, 2, 0)).reshape(CG, C)
  lamb2 = jnp.transpose(lambda_B, (1, 0, 2)).reshape(M, CG)
  x2 = x.reshape(N, 1).astype(jnp.int32)

  logtot, post = _tc_dense(sums, cnts, x2, lamqt, lamb2)
  return (logtot, post.reshape(N, C, G))
